# baseline (device time: 352875 ns/iter reference)
import jax
import jax.numpy as jnp
from jax import lax
from jax.experimental import pallas as pl
from jax.experimental.pallas import tpu as pltpu

N_DEV = 16
M_CHUNK = 256


def _sigma(r):
    r = r % N_DEV
    col = r // 4
    t = r % 4
    z = jnp.where(col % 2 == 0, t, 3 - t)
    return 4 * z + col


def _ring_pos(p):
    col = p % 4
    z = p // 4
    t = jnp.where(col % 2 == 0, z, 3 - z)
    return 4 * col + t


def kernel(x, w_mat, scale_x, scale_w):
    m, k = x.shape
    k2, n = w_mat.shape
    nq = n // 4
    assert m == N_DEV * M_CHUNK

    def body(x_ref, w_ref, sx_ref, sw_ref, out_ref, *scratch):
        comms = scratch[0:4]
        send_sems = scratch[4:8]
        recv_sems = scratch[8:12]
        credits = scratch[12:16]

        p = lax.axis_index("i")
        r = _ring_pos(p)
        nxt = _sigma(r + 1)
        prv = _sigma(r - 1)
        dsts = (nxt, nxt, prv, prv)
        srcs = (prv, prv, nxt, nxt)
        col_lo = (0, nq, 2 * nq, 3 * nq)

        barrier_sem = pltpu.get_barrier_semaphore()
        for nbr in (prv, nxt):
            pl.semaphore_signal(
                barrier_sem, inc=1,
                device_id=(nbr,), device_id_type=pl.DeviceIdType.MESH,
            )
        pl.semaphore_wait(barrier_sem, 2)

        w_bf = w_ref[:, :].astype(jnp.bfloat16)

        def partial_for(chunk, ring):
            rows = x_ref[pl.ds(chunk * M_CHUNK, M_CHUNK), :].astype(jnp.bfloat16)
            return jnp.dot(rows, w_bf[:, col_lo[ring]:col_lo[ring] + nq],
                           preferred_element_type=jnp.float32)

        def chunk_at(ring, s):
            return _sigma(r - s - 2) if ring < 2 else _sigma(r + s + 2)

        def make_rdma(ring, s):
            send_slot = s % 2
            recv_slot = (s + 1) % 2
            return pltpu.make_async_remote_copy(
                src_ref=comms[ring].at[send_slot],
                dst_ref=comms[ring].at[recv_slot],
                send_sem=send_sems[ring].at[send_slot],
                recv_sem=recv_sems[ring].at[recv_slot],
                device_id=(dsts[ring],),
                device_id_type=pl.DeviceIdType.MESH,
            )

        def start(ring, s):
            if s >= 2:
                pl.semaphore_wait(credits[ring], 1)
            make_rdma(ring, s).start()

        def finish(ring, s, acc):
            recv_slot = (s + 1) % 2
            make_rdma(ring, s).wait()
            if s <= 12:
                pl.semaphore_signal(
                    credits[ring], inc=1,
                    device_id=(srcs[ring],),
                    device_id_type=pl.DeviceIdType.MESH,
                )
            total = comms[ring][recv_slot, :, :].astype(jnp.float32) + acc
            if s < N_DEV - 2:
                comms[ring][recv_slot, :, :] = total.astype(jnp.bfloat16)
            else:
                scale = sx_ref[0] * sw_ref[0]
                out_ref[:, col_lo[ring]:col_lo[ring] + nq] = total * scale

        for ring in range(4):
            first = _sigma(r - 1) if ring < 2 else _sigma(r + 1)
            comms[ring][0, :, :] = partial_for(first, ring).astype(jnp.bfloat16)
        start(0, 0)
        start(2, 0)

        for s in range(N_DEV - 1):
            start(1, s)
            start(3, s)
            acc = [partial_for(chunk_at(ring, s), ring) for ring in range(4)]
            finish(0, s, acc[0])
            if s < N_DEV - 2:
                start(0, s + 1)
            finish(2, s, acc[2])
            if s < N_DEV - 2:
                start(2, s + 1)
            finish(1, s, acc[1])
            finish(3, s, acc[3])

    return pl.pallas_call(
        body,
        out_shape=jax.ShapeDtypeStruct((M_CHUNK, n), jnp.float32),
        in_specs=[
            pl.BlockSpec(memory_space=pltpu.VMEM),
            pl.BlockSpec(memory_space=pltpu.VMEM),
            pl.BlockSpec(memory_space=pltpu.SMEM),
            pl.BlockSpec(memory_space=pltpu.SMEM),
        ],
        out_specs=pl.BlockSpec(memory_space=pltpu.VMEM),
        scratch_shapes=(
            [pltpu.VMEM((2, M_CHUNK, nq), jnp.bfloat16) for _ in range(4)]
            + [pltpu.SemaphoreType.DMA((2,)) for _ in range(4)]
            + [pltpu.SemaphoreType.DMA((2,)) for _ in range(4)]
            + [pltpu.SemaphoreType.REGULAR for _ in range(4)]
        ),
        compiler_params=pltpu.CompilerParams(collective_id=0),
    )(x, w_mat, scale_x, scale_w)


# device time: 251214 ns/iter; 1.4047x vs baseline; 1.4047x over previous
import jax
import jax.numpy as jnp
import numpy as np
from jax import lax
from jax.experimental import pallas as pl
from jax.experimental.pallas import tpu as pltpu

N_DEV = 16
M_CHUNK = 256

CYCLES = np.array([
    [0, 3, 2, 6, 7, 4, 8, 12, 15, 11, 10, 14, 13, 9, 5, 1],
    [0, 3, 7, 6, 2, 1, 5, 9, 13, 12, 15, 14, 10, 11, 8, 4],
    [0, 4, 5, 6, 10, 9, 8, 12, 13, 14, 15, 11, 7, 3, 2, 1],
], dtype=np.int32)
INV = np.zeros_like(CYCLES)
for _c in range(3):
    INV[_c, CYCLES[_c]] = np.arange(N_DEV, dtype=np.int32)

WIDTHS = (1408, 1408, 1280)
RING_CYCLE = (0, 0, 1, 1, 2, 2)
RING_DIR = (0, 1, 0, 1, 0, 1)
RING_W = tuple(WIDTHS[c] for c in RING_CYCLE)
RING_LO = tuple(int(v) for v in np.cumsum((0,) + RING_W[:-1]))
N_RING = 6


def kernel(x, w_mat, scale_x, scale_w):
    m, k = x.shape
    k2, n = w_mat.shape
    assert m == N_DEV * M_CHUNK and sum(RING_W) == n

    def body(x_ref, w_ref, sx_ref, sw_ref, perm_ref, inv_ref, out_ref,
             *scratch):
        comms = scratch[0:N_RING]
        send_sems = scratch[N_RING:2 * N_RING]
        recv_sems = scratch[2 * N_RING:3 * N_RING]
        credits = scratch[3 * N_RING:4 * N_RING]

        p = lax.axis_index("i")
        rpos = [inv_ref[c, p] for c in range(3)]

        def perm_at(c, idx):
            return perm_ref[c, lax.rem(idx + 4 * N_DEV, N_DEV)]

        dsts = []
        srcs = []
        for i in range(N_RING):
            c = RING_CYCLE[i]
            step = 1 if RING_DIR[i] == 0 else -1
            dsts.append(perm_at(c, rpos[c] + step))
            srcs.append(perm_at(c, rpos[c] - step))

        barrier_sem = pltpu.get_barrier_semaphore()
        for nbr in dsts + srcs:
            pl.semaphore_signal(
                barrier_sem, inc=1,
                device_id=(nbr,), device_id_type=pl.DeviceIdType.MESH,
            )
        pl.semaphore_wait(barrier_sem, 2 * N_RING)

        w_bf = w_ref[:, :].astype(jnp.bfloat16)

        def partial_for(chunk, ring):
            rows = x_ref[pl.ds(chunk * M_CHUNK, M_CHUNK), :].astype(jnp.bfloat16)
            lo = RING_LO[ring]
            return jnp.dot(rows, w_bf[:, lo:lo + RING_W[ring]],
                           preferred_element_type=jnp.float32)

        def chunk_at(ring, s):
            c = RING_CYCLE[ring]
            off = -(s + 2) if RING_DIR[ring] == 0 else (s + 2)
            return perm_at(c, rpos[c] + off)

        def make_rdma(ring, s):
            send_slot = s % 2
            recv_slot = (s + 1) % 2
            return pltpu.make_async_remote_copy(
                src_ref=comms[ring].at[send_slot],
                dst_ref=comms[ring].at[recv_slot],
                send_sem=send_sems[ring].at[send_slot],
                recv_sem=recv_sems[ring].at[recv_slot],
                device_id=(dsts[ring],),
                device_id_type=pl.DeviceIdType.MESH,
            )

        def start(ring, s):
            if s >= 2:
                pl.semaphore_wait(credits[ring], 1)
            make_rdma(ring, s).start()

        def finish(ring, s, acc):
            recv_slot = (s + 1) % 2
            make_rdma(ring, s).wait()
            if s <= 12:
                pl.semaphore_signal(
                    credits[ring], inc=1,
                    device_id=(srcs[ring],),
                    device_id_type=pl.DeviceIdType.MESH,
                )
            total = comms[ring][recv_slot, :, :].astype(jnp.float32) + acc
            if s < N_DEV - 2:
                comms[ring][recv_slot, :, :] = total.astype(jnp.bfloat16)
            else:
                scale = sx_ref[0] * sw_ref[0]
                lo = RING_LO[ring]
                out_ref[:, lo:lo + RING_W[ring]] = total * scale

        for ring in range(N_RING):
            comms[ring][0, :, :] = partial_for(
                chunk_at(ring, -1), ring).astype(jnp.bfloat16)
        for ring in range(N_RING):
            start(ring, 0)

        for s in range(N_DEV - 1):
            acc = [partial_for(chunk_at(ring, s), ring)
                   for ring in range(N_RING)]
            for ring in range(N_RING):
                finish(ring, s, acc[ring])
                if s < N_DEV - 2:
                    start(ring, s + 1)

    return pl.pallas_call(
        body,
        out_shape=jax.ShapeDtypeStruct((M_CHUNK, n), jnp.float32),
        in_specs=[
            pl.BlockSpec(memory_space=pltpu.VMEM),
            pl.BlockSpec(memory_space=pltpu.VMEM),
            pl.BlockSpec(memory_space=pltpu.SMEM),
            pl.BlockSpec(memory_space=pltpu.SMEM),
            pl.BlockSpec(memory_space=pltpu.SMEM),
            pl.BlockSpec(memory_space=pltpu.SMEM),
        ],
        out_specs=pl.BlockSpec(memory_space=pltpu.VMEM),
        scratch_shapes=(
            [pltpu.VMEM((2, M_CHUNK, RING_W[i]), jnp.bfloat16)
             for i in range(N_RING)]
            + [pltpu.SemaphoreType.DMA((2,)) for _ in range(N_RING)]
            + [pltpu.SemaphoreType.DMA((2,)) for _ in range(N_RING)]
            + [pltpu.SemaphoreType.REGULAR for _ in range(N_RING)]
        ),
        compiler_params=pltpu.CompilerParams(collective_id=0),
    )(x, w_mat, scale_x, scale_w, jnp.asarray(CYCLES), jnp.asarray(INV))
